# baseline (speedup 1.0000x reference)
import functools

import jax
import jax.numpy as jnp
from jax import lax
from jax.experimental import pallas as pl
from jax.experimental.pallas import tpu as pltpu
from jax.experimental.pallas import tpu_sc as plsc

BATCH = 16384
CF_DIM = 64
CD = 256
ALPHA = 0.5

NC = 2
NS = 16
NW = NC * NS
BPW = BATCH // NW
SUB = 128
NSUB = BPW // SUB


@functools.cache
def _make_sc_gather():
    mesh = plsc.VectorSubcoreMesh(core_axis_name="c", subcore_axis_name="s",
                                  num_cores=NC, num_subcores=NS)

    @functools.partial(
        pl.kernel,
        out_type=[
            jax.ShapeDtypeStruct((BATCH, CF_DIM), jnp.float32),
            jax.ShapeDtypeStruct((BATCH, CF_DIM), jnp.float32),
            jax.ShapeDtypeStruct((BATCH, CD), jnp.float32),
            jax.ShapeDtypeStruct((BATCH, CD), jnp.float32),
        ],
        mesh=mesh,
        compiler_params=pltpu.CompilerParams(use_tc_tiling_on_sc=False),
        scratch_types=[
            pltpu.VMEM((BPW,), jnp.int32),
            pltpu.VMEM((BPW,), jnp.int32),
            pltpu.VMEM((SUB, CF_DIM), jnp.float32),
            pltpu.VMEM((SUB, CF_DIM), jnp.float32),
            pltpu.VMEM((SUB, CD), jnp.float32),
            pltpu.VMEM((SUB, CD), jnp.float32),
            pltpu.SemaphoreType.DMA,
        ],
    )
    def _sc_gather(uids, iids, ucf, icf, uprof, icont,
                   ucf_out, icf_out, uprof_out, icont_out,
                   uid_v, iid_v, ubuf64, ibuf64, ubuf256, ibuf256, sem):
        wid = lax.axis_index("s") * NC + lax.axis_index("c")
        base = wid * BPW
        pltpu.sync_copy(uids.at[pl.ds(base, BPW)], uid_v)
        pltpu.sync_copy(iids.at[pl.ds(base, BPW)], iid_v)
        for c in range(NSUB):
            o = c * SUB
            pltpu.async_copy(ucf.at[uid_v.at[pl.ds(o, SUB)]], ubuf64, sem).wait()
            pltpu.async_copy(icf.at[iid_v.at[pl.ds(o, SUB)]], ibuf64, sem).wait()
            pltpu.async_copy(uprof.at[uid_v.at[pl.ds(o, SUB)]], ubuf256, sem).wait()
            pltpu.async_copy(icont.at[iid_v.at[pl.ds(o, SUB)]], ibuf256, sem).wait()
            pltpu.sync_copy(ubuf64, ucf_out.at[pl.ds(base + o, SUB)])
            pltpu.sync_copy(ibuf64, icf_out.at[pl.ds(base + o, SUB)])
            pltpu.sync_copy(ubuf256, uprof_out.at[pl.ds(base + o, SUB)])
            pltpu.sync_copy(ibuf256, icont_out.at[pl.ds(base + o, SUB)])

    return _sc_gather


BLK = 1024


def _tc_body(ucf_ref, icf_ref, uprof_ref, icont_ref,
             w_ref, b_ref, g_ref, beta_ref, out_ref):
    u = uprof_ref[...]
    h = jnp.dot(u, w_ref[...], preferred_element_type=jnp.float32)
    h = h + b_ref[...]
    mu = jnp.mean(h, axis=1, keepdims=True)
    var = jnp.mean((h - mu) * (h - mu), axis=1, keepdims=True)
    hn = (h - mu) * lax.rsqrt(var + 1e-5) * g_ref[...] + beta_ref[...]
    hg = hn * 0.5 * (1.0 + lax.erf(hn * 0.7071067811865476))
    content = jnp.sum(hg * icont_ref[...], axis=1)
    cf = jnp.sum(ucf_ref[...] * icf_ref[...], axis=1)
    out_ref[...] = ALPHA * cf + (1.0 - ALPHA) * content


_tc_score = pl.pallas_call(
    _tc_body,
    grid=(BATCH // BLK,),
    in_specs=[
        pl.BlockSpec((BLK, CF_DIM), lambda i: (i, 0)),
        pl.BlockSpec((BLK, CF_DIM), lambda i: (i, 0)),
        pl.BlockSpec((BLK, CD), lambda i: (i, 0)),
        pl.BlockSpec((BLK, CD), lambda i: (i, 0)),
        pl.BlockSpec((CD, CD), lambda i: (0, 0)),
        pl.BlockSpec((1, CD), lambda i: (0, 0)),
        pl.BlockSpec((1, CD), lambda i: (0, 0)),
        pl.BlockSpec((1, CD), lambda i: (0, 0)),
    ],
    out_specs=pl.BlockSpec((BLK,), lambda i: (i,)),
    out_shape=jax.ShapeDtypeStruct((BATCH,), jnp.float32),
)


def kernel(user_ids, item_ids, user_cf_weight, item_cf_weight,
           raw_user_profiles, article_content_embeddings,
           proj_W, proj_b, ln_gamma, ln_beta):
    ucf_g, icf_g, uprof_g, icont_g = _make_sc_gather()(
        user_ids, item_ids, user_cf_weight, item_cf_weight,
        raw_user_profiles, article_content_embeddings)
    return _tc_score(ucf_g, icf_g, uprof_g, icont_g,
                     proj_W, proj_b.reshape(1, CD), ln_gamma.reshape(1, CD),
                     ln_beta.reshape(1, CD))


# speedup vs baseline: 1.7719x; 1.7719x over previous
import functools

import jax
import jax.numpy as jnp
from jax import lax
from jax.experimental import pallas as pl
from jax.experimental.pallas import tpu as pltpu
from jax.experimental.pallas import tpu_sc as plsc

BATCH = 16384
CF_DIM = 64
CD = 256
ALPHA = 0.5

NC = 2
NS = 16
NW = NC * NS
BPW = BATCH // NW
LANES = 16

CF_SUB = 128
CF_NSUB = BPW // CF_SUB
PR_SUB = 64
PR_NSUB = BPW // PR_SUB


def _mesh():
    return plsc.VectorSubcoreMesh(core_axis_name="c", subcore_axis_name="s",
                                  num_cores=NC, num_subcores=NS)


@functools.cache
def _make_sc_cf():
    @functools.partial(
        pl.kernel,
        out_type=jax.ShapeDtypeStruct((BATCH,), jnp.float32),
        mesh=_mesh(),
        compiler_params=pltpu.CompilerParams(use_tc_tiling_on_sc=False,
                                             needs_layout_passes=False),
        scratch_types=[
            pltpu.VMEM((BPW,), jnp.int32),
            pltpu.VMEM((BPW,), jnp.int32),
            pltpu.VMEM((CF_SUB, CF_DIM), jnp.float32),
            pltpu.VMEM((CF_SUB, CF_DIM), jnp.float32),
            pltpu.VMEM((BPW,), jnp.float32),
            pltpu.SemaphoreType.DMA,
            pltpu.SemaphoreType.DMA,
        ],
    )
    def _sc_cf(uids, iids, ucf, icf, cf_out,
               uid_v, iid_v, ubuf, ibuf, score_v, sem_u, sem_i):
        wid = lax.axis_index("s") * NC + lax.axis_index("c")
        base = wid * BPW
        pltpu.sync_copy(uids.at[pl.ds(base, BPW)], uid_v)
        pltpu.sync_copy(iids.at[pl.ds(base, BPW)], iid_v)
        for c in range(CF_NSUB):
            o = c * CF_SUB
            cu = pltpu.async_copy(ucf.at[uid_v.at[pl.ds(o, CF_SUB)]], ubuf, sem_u)
            ci = pltpu.async_copy(icf.at[iid_v.at[pl.ds(o, CF_SUB)]], ibuf, sem_i)
            cu.wait()
            ci.wait()
            for p0 in range(0, CF_SUB, LANES):
                rows = lax.iota(jnp.int32, LANES) + p0

                def dot_body(d, acc):
                    cols = jnp.full((LANES,), d, jnp.int32)
                    uv = plsc.load_gather(ubuf, [rows, cols])
                    iv = plsc.load_gather(ibuf, [rows, cols])
                    return acc + uv * iv

                acc = lax.fori_loop(0, CF_DIM, dot_body,
                                    jnp.zeros((LANES,), jnp.float32))
                score_v[pl.ds(o + p0, LANES)] = acc
        pltpu.sync_copy(score_v, cf_out.at[pl.ds(base, BPW)])

    return _sc_cf


@functools.cache
def _make_sc_gather():
    @functools.partial(
        pl.kernel,
        out_type=[
            jax.ShapeDtypeStruct((BATCH, CD), jnp.float32),
            jax.ShapeDtypeStruct((BATCH, CD), jnp.float32),
        ],
        mesh=_mesh(),
        scratch_types=[
            pltpu.VMEM((BPW,), jnp.int32),
            pltpu.VMEM((BPW,), jnp.int32),
            pltpu.VMEM((2, PR_SUB, CD), jnp.float32),
            pltpu.VMEM((2, PR_SUB, CD), jnp.float32),
            pltpu.SemaphoreType.DMA,
            pltpu.SemaphoreType.DMA,
            pltpu.SemaphoreType.DMA,
            pltpu.SemaphoreType.DMA,
        ],
    )
    def _sc_gather(uids, iids, uprof, icont,
                   uprof_out, icont_out,
                   uid_v, iid_v, ubuf, ibuf, su0, su1, si0, si1):
        wid = lax.axis_index("s") * NC + lax.axis_index("c")
        base = wid * BPW
        pltpu.sync_copy(uids.at[pl.ds(base, BPW)], uid_v)
        pltpu.sync_copy(iids.at[pl.ds(base, BPW)], iid_v)
        sem_u = (su0, su1)
        sem_i = (si0, si1)

        def fire(c):
            o = c * PR_SUB
            s = c % 2
            cu = pltpu.async_copy(uprof.at[uid_v.at[pl.ds(o, PR_SUB)]],
                                  ubuf.at[s], sem_u[s])
            ci = pltpu.async_copy(icont.at[iid_v.at[pl.ds(o, PR_SUB)]],
                                  ibuf.at[s], sem_i[s])
            return cu, ci

        pend = fire(0)
        for c in range(PR_NSUB):
            cu, ci = pend
            if c + 1 < PR_NSUB:
                nxt = fire(c + 1)
            cu.wait()
            ci.wait()
            o = c * PR_SUB
            s = c % 2
            pltpu.sync_copy(ubuf.at[s], uprof_out.at[pl.ds(base + o, PR_SUB)])
            pltpu.sync_copy(ibuf.at[s], icont_out.at[pl.ds(base + o, PR_SUB)])
            if c + 1 < PR_NSUB:
                pend = nxt

    return _sc_gather


BLK = 1024


def _tc_body(cf_ref, uprof_ref, icont_ref, w_ref, b_ref, g_ref, beta_ref,
             out_ref):
    u = uprof_ref[...]
    h = jnp.dot(u, w_ref[...], preferred_element_type=jnp.float32)
    h = h + b_ref[...]
    mu = jnp.mean(h, axis=1, keepdims=True)
    var = jnp.mean((h - mu) * (h - mu), axis=1, keepdims=True)
    hn = (h - mu) * lax.rsqrt(var + 1e-5) * g_ref[...] + beta_ref[...]
    hg = hn * 0.5 * (1.0 + lax.erf(hn * 0.7071067811865476))
    content = jnp.sum(hg * icont_ref[...], axis=1)
    out_ref[...] = ALPHA * cf_ref[...] + (1.0 - ALPHA) * content


_tc_score = pl.pallas_call(
    _tc_body,
    grid=(BATCH // BLK,),
    in_specs=[
        pl.BlockSpec((BLK,), lambda i: (i,)),
        pl.BlockSpec((BLK, CD), lambda i: (i, 0)),
        pl.BlockSpec((BLK, CD), lambda i: (i, 0)),
        pl.BlockSpec((CD, CD), lambda i: (0, 0)),
        pl.BlockSpec((1, CD), lambda i: (0, 0)),
        pl.BlockSpec((1, CD), lambda i: (0, 0)),
        pl.BlockSpec((1, CD), lambda i: (0, 0)),
    ],
    out_specs=pl.BlockSpec((BLK,), lambda i: (i,)),
    out_shape=jax.ShapeDtypeStruct((BATCH,), jnp.float32),
)


def kernel(user_ids, item_ids, user_cf_weight, item_cf_weight,
           raw_user_profiles, article_content_embeddings,
           proj_W, proj_b, ln_gamma, ln_beta):
    cf = _make_sc_cf()(user_ids, item_ids, user_cf_weight, item_cf_weight)
    uprof_g, icont_g = _make_sc_gather()(
        user_ids, item_ids, raw_user_profiles, article_content_embeddings)
    return _tc_score(cf, uprof_g, icont_g,
                     proj_W, proj_b.reshape(1, CD), ln_gamma.reshape(1, CD),
                     ln_beta.reshape(1, CD))


# speedup vs baseline: 1.8353x; 1.0358x over previous
import functools

import jax
import jax.numpy as jnp
from jax import lax
from jax.experimental import pallas as pl
from jax.experimental.pallas import tpu as pltpu
from jax.experimental.pallas import tpu_sc as plsc

BATCH = 16384
CF_DIM = 64
CD = 256
ALPHA = 0.5

NC = 2
NS = 16
NW = NC * NS
BPW = BATCH // NW
LANES = 16

CF_SUB = 128
CF_NSUB = BPW // CF_SUB
PR_SUB = 64
PR_NSUB = BPW // PR_SUB


def _mesh():
    return plsc.VectorSubcoreMesh(core_axis_name="c", subcore_axis_name="s",
                                  num_cores=NC, num_subcores=NS)


@functools.cache
def _make_sc_cf():
    @functools.partial(
        pl.kernel,
        out_type=jax.ShapeDtypeStruct((BATCH,), jnp.float32),
        mesh=_mesh(),
        compiler_params=pltpu.CompilerParams(needs_layout_passes=False),
        scratch_types=[
            pltpu.VMEM((BPW,), jnp.int32),
            pltpu.VMEM((BPW,), jnp.int32),
            pltpu.VMEM((CF_SUB, 2 * CF_DIM), jnp.float32),
            pltpu.VMEM((CF_SUB, 2 * CF_DIM), jnp.float32),
            pltpu.VMEM((BPW,), jnp.float32),
            pltpu.SemaphoreType.DMA,
            pltpu.SemaphoreType.DMA,
        ],
    )
    def _sc_cf(uids, iids, cfcat, cf_out,
               uid_v, iid_v, ubuf, ibuf, score_v, sem_u, sem_i):
        wid = lax.axis_index("s") * NC + lax.axis_index("c")
        base = wid * BPW
        pltpu.sync_copy(uids.at[pl.ds(base, BPW)], uid_v)
        pltpu.sync_copy(iids.at[pl.ds(base, BPW)], iid_v)
        for c in range(CF_NSUB):
            o = c * CF_SUB
            cu = pltpu.async_copy(cfcat.at[uid_v.at[pl.ds(o, CF_SUB)]], ubuf, sem_u)
            ci = pltpu.async_copy(cfcat.at[iid_v.at[pl.ds(o, CF_SUB)]], ibuf, sem_i)
            cu.wait()
            ci.wait()
            for p0 in range(0, CF_SUB, LANES):
                rows = lax.iota(jnp.int32, LANES) + p0

                def dot_body(d, acc):
                    ucols = jnp.full((LANES,), d, jnp.int32)
                    icols = ucols + CF_DIM
                    uv = plsc.load_gather(ubuf, [rows, ucols])
                    iv = plsc.load_gather(ibuf, [rows, icols])
                    return acc + uv * iv

                acc = lax.fori_loop(0, CF_DIM, dot_body,
                                    jnp.zeros((LANES,), jnp.float32))
                score_v[pl.ds(o + p0, LANES)] = acc
        pltpu.sync_copy(score_v, cf_out.at[pl.ds(base, BPW)])

    return _sc_cf


@functools.cache
def _make_sc_gather():
    @functools.partial(
        pl.kernel,
        out_type=[
            jax.ShapeDtypeStruct((BATCH, CD), jnp.float32),
            jax.ShapeDtypeStruct((BATCH, CD), jnp.float32),
        ],
        mesh=_mesh(),
        scratch_types=[
            pltpu.VMEM((BPW,), jnp.int32),
            pltpu.VMEM((BPW,), jnp.int32),
            pltpu.VMEM((2, PR_SUB, CD), jnp.float32),
            pltpu.VMEM((2, PR_SUB, CD), jnp.float32),
            pltpu.SemaphoreType.DMA,
            pltpu.SemaphoreType.DMA,
            pltpu.SemaphoreType.DMA,
            pltpu.SemaphoreType.DMA,
        ],
    )
    def _sc_gather(uids, iids, uprof, icont,
                   uprof_out, icont_out,
                   uid_v, iid_v, ubuf, ibuf, su0, su1, si0, si1):
        wid = lax.axis_index("s") * NC + lax.axis_index("c")
        base = wid * BPW
        pltpu.sync_copy(uids.at[pl.ds(base, BPW)], uid_v)
        pltpu.sync_copy(iids.at[pl.ds(base, BPW)], iid_v)
        sem_u = (su0, su1)
        sem_i = (si0, si1)

        def fire(c):
            o = c * PR_SUB
            s = c % 2
            cu = pltpu.async_copy(uprof.at[uid_v.at[pl.ds(o, PR_SUB)]],
                                  ubuf.at[s], sem_u[s])
            ci = pltpu.async_copy(icont.at[iid_v.at[pl.ds(o, PR_SUB)]],
                                  ibuf.at[s], sem_i[s])
            return cu, ci

        pend = fire(0)
        for c in range(PR_NSUB):
            cu, ci = pend
            if c + 1 < PR_NSUB:
                nxt = fire(c + 1)
            cu.wait()
            ci.wait()
            o = c * PR_SUB
            s = c % 2
            pltpu.sync_copy(ubuf.at[s], uprof_out.at[pl.ds(base + o, PR_SUB)])
            pltpu.sync_copy(ibuf.at[s], icont_out.at[pl.ds(base + o, PR_SUB)])
            if c + 1 < PR_NSUB:
                pend = nxt

    return _sc_gather


BLK = 1024


def _tc_body(cf_ref, uprof_ref, icont_ref, w_ref, b_ref, g_ref, beta_ref,
             out_ref):
    u = uprof_ref[...]
    h = jnp.dot(u, w_ref[...], preferred_element_type=jnp.float32)
    h = h + b_ref[...]
    mu = jnp.mean(h, axis=1, keepdims=True)
    var = jnp.mean((h - mu) * (h - mu), axis=1, keepdims=True)
    hn = (h - mu) * lax.rsqrt(var + 1e-5) * g_ref[...] + beta_ref[...]
    hg = hn * 0.5 * (1.0 + lax.erf(hn * 0.7071067811865476))
    content = jnp.sum(hg * icont_ref[...], axis=1)
    out_ref[...] = ALPHA * cf_ref[...] + (1.0 - ALPHA) * content


_tc_score = pl.pallas_call(
    _tc_body,
    grid=(BATCH // BLK,),
    in_specs=[
        pl.BlockSpec((BLK,), lambda i: (i,)),
        pl.BlockSpec((BLK, CD), lambda i: (i, 0)),
        pl.BlockSpec((BLK, CD), lambda i: (i, 0)),
        pl.BlockSpec((CD, CD), lambda i: (0, 0)),
        pl.BlockSpec((1, CD), lambda i: (0, 0)),
        pl.BlockSpec((1, CD), lambda i: (0, 0)),
        pl.BlockSpec((1, CD), lambda i: (0, 0)),
    ],
    out_specs=pl.BlockSpec((BLK,), lambda i: (i,)),
    out_shape=jax.ShapeDtypeStruct((BATCH,), jnp.float32),
)


def kernel(user_ids, item_ids, user_cf_weight, item_cf_weight,
           raw_user_profiles, article_content_embeddings,
           proj_W, proj_b, ln_gamma, ln_beta):
    cfcat = jnp.concatenate([user_cf_weight, item_cf_weight], axis=1)
    cf = _make_sc_cf()(user_ids, item_ids, cfcat)
    uprof_g, icont_g = _make_sc_gather()(
        user_ids, item_ids, raw_user_profiles, article_content_embeddings)
    return _tc_score(cf, uprof_g, icont_g,
                     proj_W, proj_b.reshape(1, CD), ln_gamma.reshape(1, CD),
                     ln_beta.reshape(1, CD))


# speedup vs baseline: 2.2773x; 1.2408x over previous
import functools

import jax
import jax.numpy as jnp
from jax import lax
from jax.experimental import pallas as pl
from jax.experimental.pallas import tpu as pltpu
from jax.experimental.pallas import tpu_sc as plsc

BATCH = 16384
CF_DIM = 64
CD = 256
ALPHA = 0.5

NC = 2
NS = 16
NW = NC * NS
BPW = BATCH // NW
LANES = 16

CF_SUB = 128
CF_NSUB = BPW // CF_SUB
PR_SUB = 64
PR_NSUB = BPW // PR_SUB


def _mesh():
    return plsc.VectorSubcoreMesh(core_axis_name="c", subcore_axis_name="s",
                                  num_cores=NC, num_subcores=NS)


@functools.cache
def _make_sc_cf():
    @functools.partial(
        pl.kernel,
        out_type=jax.ShapeDtypeStruct((BATCH,), jnp.float32),
        mesh=_mesh(),
        compiler_params=pltpu.CompilerParams(needs_layout_passes=False),
        scratch_types=[
            pltpu.VMEM((BPW,), jnp.int32),
            pltpu.VMEM((BPW,), jnp.int32),
            pltpu.VMEM((CF_SUB, 2 * CF_DIM), jnp.float32),
            pltpu.VMEM((CF_SUB, 2 * CF_DIM), jnp.float32),
            pltpu.VMEM((BPW,), jnp.float32),
            pltpu.SemaphoreType.DMA,
            pltpu.SemaphoreType.DMA,
        ],
    )
    def _sc_cf(uids, iids, cfcat, cf_out,
               uid_v, iid_v, ubuf, ibuf, score_v, sem_u, sem_i):
        wid = lax.axis_index("s") * NC + lax.axis_index("c")
        base = wid * BPW
        pltpu.sync_copy(uids.at[pl.ds(base, BPW)], uid_v)
        pltpu.sync_copy(iids.at[pl.ds(base, BPW)], iid_v)
        for c in range(CF_NSUB):
            o = c * CF_SUB
            cu = pltpu.async_copy(cfcat.at[uid_v.at[pl.ds(o, CF_SUB)]], ubuf, sem_u)
            ci = pltpu.async_copy(cfcat.at[iid_v.at[pl.ds(o, CF_SUB)]], ibuf, sem_i)
            cu.wait()
            ci.wait()
            for p0 in range(0, CF_SUB, LANES):
                rows = lax.iota(jnp.int32, LANES) + p0

                def dot_body(k, accs):
                    d0 = k * 4
                    new = []
                    for j in range(4):
                        ucols = jnp.full((LANES,), d0 + j, jnp.int32)
                        uv = plsc.load_gather(ubuf, [rows, ucols])
                        iv = plsc.load_gather(ibuf, [rows, ucols + CF_DIM])
                        new.append(accs[j] + uv * iv)
                    return tuple(new)

                z = jnp.zeros((LANES,), jnp.float32)
                a0, a1, a2, a3 = lax.fori_loop(0, CF_DIM // 4, dot_body,
                                               (z, z, z, z))
                score_v[pl.ds(o + p0, LANES)] = (a0 + a1) + (a2 + a3)
        pltpu.sync_copy(score_v, cf_out.at[pl.ds(base, BPW)])

    return _sc_cf


@functools.cache
def _make_sc_gather():
    @functools.partial(
        pl.kernel,
        out_type=[
            jax.ShapeDtypeStruct((BATCH, CD), jnp.float32),
            jax.ShapeDtypeStruct((BATCH, CD), jnp.float32),
        ],
        mesh=_mesh(),
        scratch_types=[
            pltpu.VMEM((BPW,), jnp.int32),
            pltpu.VMEM((BPW,), jnp.int32),
            pltpu.VMEM((2, PR_SUB, CD), jnp.float32),
            pltpu.VMEM((2, PR_SUB, CD), jnp.float32),
            pltpu.SemaphoreType.DMA,
            pltpu.SemaphoreType.DMA,
            pltpu.SemaphoreType.DMA,
            pltpu.SemaphoreType.DMA,
        ],
    )
    def _sc_gather(uids, iids, uprof, icont,
                   uprof_out, icont_out,
                   uid_v, iid_v, ubuf, ibuf, su0, su1, si0, si1):
        wid = lax.axis_index("s") * NC + lax.axis_index("c")
        base = wid * BPW
        pltpu.sync_copy(uids.at[pl.ds(base, BPW)], uid_v)
        pltpu.sync_copy(iids.at[pl.ds(base, BPW)], iid_v)
        sem_u = (su0, su1)
        sem_i = (si0, si1)

        def fire(c):
            o = c * PR_SUB
            s = c % 2
            cu = pltpu.async_copy(uprof.at[uid_v.at[pl.ds(o, PR_SUB)]],
                                  ubuf.at[s], sem_u[s])
            ci = pltpu.async_copy(icont.at[iid_v.at[pl.ds(o, PR_SUB)]],
                                  ibuf.at[s], sem_i[s])
            return cu, ci

        pend = fire(0)
        for c in range(PR_NSUB):
            cu, ci = pend
            if c + 1 < PR_NSUB:
                nxt = fire(c + 1)
            cu.wait()
            ci.wait()
            o = c * PR_SUB
            s = c % 2
            pltpu.sync_copy(ubuf.at[s], uprof_out.at[pl.ds(base + o, PR_SUB)])
            pltpu.sync_copy(ibuf.at[s], icont_out.at[pl.ds(base + o, PR_SUB)])
            if c + 1 < PR_NSUB:
                pend = nxt

    return _sc_gather


N_ROWS = 100000
N_PAD = 100096
PREP_R = 5888


def _tc_prep_body(ut_ref, it_ref, out_ref):
    out_ref[...] = jnp.concatenate([ut_ref[...].T, it_ref[...].T], axis=1)


_tc_prep = pl.pallas_call(
    _tc_prep_body,
    grid=(N_PAD // PREP_R,),
    in_specs=[
        pl.BlockSpec((CF_DIM, PREP_R), lambda i: (0, i)),
        pl.BlockSpec((CF_DIM, PREP_R), lambda i: (0, i)),
    ],
    out_specs=pl.BlockSpec((PREP_R, 2 * CF_DIM), lambda i: (i, 0)),
    out_shape=jax.ShapeDtypeStruct((N_PAD, 2 * CF_DIM), jnp.float32),
)


BLK = 1024


def _tc_body(cf_ref, uprof_ref, icont_ref, w_ref, b_ref, g_ref, beta_ref,
             out_ref):
    u = uprof_ref[...]
    h = jnp.dot(u, w_ref[...], preferred_element_type=jnp.float32)
    h = h + b_ref[...]
    mu = jnp.mean(h, axis=1, keepdims=True)
    var = jnp.mean((h - mu) * (h - mu), axis=1, keepdims=True)
    hn = (h - mu) * lax.rsqrt(var + 1e-5) * g_ref[...] + beta_ref[...]
    hg = hn * 0.5 * (1.0 + lax.erf(hn * 0.7071067811865476))
    content = jnp.sum(hg * icont_ref[...], axis=1)
    out_ref[...] = ALPHA * cf_ref[...] + (1.0 - ALPHA) * content


_tc_score = pl.pallas_call(
    _tc_body,
    grid=(BATCH // BLK,),
    in_specs=[
        pl.BlockSpec((BLK,), lambda i: (i,)),
        pl.BlockSpec((BLK, CD), lambda i: (i, 0)),
        pl.BlockSpec((BLK, CD), lambda i: (i, 0)),
        pl.BlockSpec((CD, CD), lambda i: (0, 0)),
        pl.BlockSpec((1, CD), lambda i: (0, 0)),
        pl.BlockSpec((1, CD), lambda i: (0, 0)),
        pl.BlockSpec((1, CD), lambda i: (0, 0)),
    ],
    out_specs=pl.BlockSpec((BLK,), lambda i: (i,)),
    out_shape=jax.ShapeDtypeStruct((BATCH,), jnp.float32),
)


def kernel(user_ids, item_ids, user_cf_weight, item_cf_weight,
           raw_user_profiles, article_content_embeddings,
           proj_W, proj_b, ln_gamma, ln_beta):
    cfcat = _tc_prep(user_cf_weight.T, item_cf_weight.T)
    cf = _make_sc_cf()(user_ids, item_ids, cfcat)
    uprof_g, icont_g = _make_sc_gather()(
        user_ids, item_ids, raw_user_profiles, article_content_embeddings)
    return _tc_score(cf, uprof_g, icont_g,
                     proj_W, proj_b.reshape(1, CD), ln_gamma.reshape(1, CD),
                     ln_beta.reshape(1, CD))


# speedup vs baseline: 2.6398x; 1.1592x over previous
import functools

import jax
import jax.numpy as jnp
from jax import lax
from jax.experimental import pallas as pl
from jax.experimental.pallas import tpu as pltpu
from jax.experimental.pallas import tpu_sc as plsc

BATCH = 16384
CF_DIM = 64
CD = 256
ALPHA = 0.5

NC = 2
NS = 16
NW = NC * NS
BPW = BATCH // NW

CF_SUB = 128
CF_NSUB = BPW // CF_SUB
PR_SUB = 64
PR_NSUB = BPW // PR_SUB

N_ROWS = 100000
N_PAD = 100096
PREP_R = 5888


def _mesh():
    return plsc.VectorSubcoreMesh(core_axis_name="c", subcore_axis_name="s",
                                  num_cores=NC, num_subcores=NS)


@functools.cache
def _make_sc_cfgather():
    @functools.partial(
        pl.kernel,
        out_type=[
            jax.ShapeDtypeStruct((BATCH, 2 * CF_DIM), jnp.float32),
            jax.ShapeDtypeStruct((BATCH, 2 * CF_DIM), jnp.float32),
        ],
        mesh=_mesh(),
        scratch_types=[
            pltpu.VMEM((BPW,), jnp.int32),
            pltpu.VMEM((BPW,), jnp.int32),
            pltpu.VMEM((2, CF_SUB, 2 * CF_DIM), jnp.float32),
            pltpu.VMEM((2, CF_SUB, 2 * CF_DIM), jnp.float32),
            pltpu.SemaphoreType.DMA,
            pltpu.SemaphoreType.DMA,
            pltpu.SemaphoreType.DMA,
            pltpu.SemaphoreType.DMA,
        ],
    )
    def _sc_cfgather(uids, iids, cfcat, ucf_out, icf_out,
                     uid_v, iid_v, ubuf, ibuf, su0, su1, si0, si1):
        wid = lax.axis_index("s") * NC + lax.axis_index("c")
        base = wid * BPW
        pltpu.sync_copy(uids.at[pl.ds(base, BPW)], uid_v)
        pltpu.sync_copy(iids.at[pl.ds(base, BPW)], iid_v)
        sem_u = (su0, su1)
        sem_i = (si0, si1)

        def fire(c):
            o = c * CF_SUB
            s = c % 2
            cu = pltpu.async_copy(cfcat.at[uid_v.at[pl.ds(o, CF_SUB)]],
                                  ubuf.at[s], sem_u[s])
            ci = pltpu.async_copy(cfcat.at[iid_v.at[pl.ds(o, CF_SUB)]],
                                  ibuf.at[s], sem_i[s])
            return cu, ci

        pend = fire(0)
        for c in range(CF_NSUB):
            cu, ci = pend
            if c + 1 < CF_NSUB:
                nxt = fire(c + 1)
            cu.wait()
            ci.wait()
            o = c * CF_SUB
            s = c % 2
            pltpu.sync_copy(ubuf.at[s], ucf_out.at[pl.ds(base + o, CF_SUB)])
            pltpu.sync_copy(ibuf.at[s], icf_out.at[pl.ds(base + o, CF_SUB)])
            if c + 1 < CF_NSUB:
                pend = nxt

    return _sc_cfgather


@functools.cache
def _make_sc_gather():
    @functools.partial(
        pl.kernel,
        out_type=[
            jax.ShapeDtypeStruct((BATCH, CD), jnp.float32),
            jax.ShapeDtypeStruct((BATCH, CD), jnp.float32),
        ],
        mesh=_mesh(),
        scratch_types=[
            pltpu.VMEM((BPW,), jnp.int32),
            pltpu.VMEM((BPW,), jnp.int32),
            pltpu.VMEM((2, PR_SUB, CD), jnp.float32),
            pltpu.VMEM((2, PR_SUB, CD), jnp.float32),
            pltpu.SemaphoreType.DMA,
            pltpu.SemaphoreType.DMA,
            pltpu.SemaphoreType.DMA,
            pltpu.SemaphoreType.DMA,
        ],
    )
    def _sc_gather(uids, iids, uprof, icont,
                   uprof_out, icont_out,
                   uid_v, iid_v, ubuf, ibuf, su0, su1, si0, si1):
        wid = lax.axis_index("s") * NC + lax.axis_index("c")
        base = wid * BPW
        pltpu.sync_copy(uids.at[pl.ds(base, BPW)], uid_v)
        pltpu.sync_copy(iids.at[pl.ds(base, BPW)], iid_v)
        sem_u = (su0, su1)
        sem_i = (si0, si1)

        def fire(c):
            o = c * PR_SUB
            s = c % 2
            cu = pltpu.async_copy(uprof.at[uid_v.at[pl.ds(o, PR_SUB)]],
                                  ubuf.at[s], sem_u[s])
            ci = pltpu.async_copy(icont.at[iid_v.at[pl.ds(o, PR_SUB)]],
                                  ibuf.at[s], sem_i[s])
            return cu, ci

        pend = fire(0)
        for c in range(PR_NSUB):
            cu, ci = pend
            if c + 1 < PR_NSUB:
                nxt = fire(c + 1)
            cu.wait()
            ci.wait()
            o = c * PR_SUB
            s = c % 2
            pltpu.sync_copy(ubuf.at[s], uprof_out.at[pl.ds(base + o, PR_SUB)])
            pltpu.sync_copy(ibuf.at[s], icont_out.at[pl.ds(base + o, PR_SUB)])
            if c + 1 < PR_NSUB:
                pend = nxt

    return _sc_gather


def _tc_prep_body(ut_ref, it_ref, out_ref):
    out_ref[...] = jnp.concatenate([ut_ref[...].T, it_ref[...].T], axis=1)


_tc_prep = pl.pallas_call(
    _tc_prep_body,
    grid=(N_PAD // PREP_R,),
    in_specs=[
        pl.BlockSpec((CF_DIM, PREP_R), lambda i: (0, i)),
        pl.BlockSpec((CF_DIM, PREP_R), lambda i: (0, i)),
    ],
    out_specs=pl.BlockSpec((PREP_R, 2 * CF_DIM), lambda i: (i, 0)),
    out_shape=jax.ShapeDtypeStruct((N_PAD, 2 * CF_DIM), jnp.float32),
)


BLK = 1024


def _tc_body(ucf_ref, icf_ref, uprof_ref, icont_ref, w_ref, b_ref, g_ref,
             beta_ref, out_ref):
    u = uprof_ref[...]
    h = jnp.dot(u, w_ref[...], preferred_element_type=jnp.float32)
    h = h + b_ref[...]
    mu = jnp.mean(h, axis=1, keepdims=True)
    var = jnp.mean((h - mu) * (h - mu), axis=1, keepdims=True)
    hn = (h - mu) * lax.rsqrt(var + 1e-5) * g_ref[...] + beta_ref[...]
    hg = hn * 0.5 * (1.0 + lax.erf(hn * 0.7071067811865476))
    content = jnp.sum(hg * icont_ref[...], axis=1)
    cf = jnp.sum(ucf_ref[:, :CF_DIM] * icf_ref[:, CF_DIM:], axis=1)
    out_ref[...] = ALPHA * cf + (1.0 - ALPHA) * content


_tc_score = pl.pallas_call(
    _tc_body,
    grid=(BATCH // BLK,),
    in_specs=[
        pl.BlockSpec((BLK, 2 * CF_DIM), lambda i: (i, 0)),
        pl.BlockSpec((BLK, 2 * CF_DIM), lambda i: (i, 0)),
        pl.BlockSpec((BLK, CD), lambda i: (i, 0)),
        pl.BlockSpec((BLK, CD), lambda i: (i, 0)),
        pl.BlockSpec((CD, CD), lambda i: (0, 0)),
        pl.BlockSpec((1, CD), lambda i: (0, 0)),
        pl.BlockSpec((1, CD), lambda i: (0, 0)),
        pl.BlockSpec((1, CD), lambda i: (0, 0)),
    ],
    out_specs=pl.BlockSpec((BLK,), lambda i: (i,)),
    out_shape=jax.ShapeDtypeStruct((BATCH,), jnp.float32),
)


def kernel(user_ids, item_ids, user_cf_weight, item_cf_weight,
           raw_user_profiles, article_content_embeddings,
           proj_W, proj_b, ln_gamma, ln_beta):
    uprof_g, icont_g = _make_sc_gather()(
        user_ids, item_ids, raw_user_profiles, article_content_embeddings)
    cfcat = _tc_prep(user_cf_weight.T, item_cf_weight.T)
    ucf_g, icf_g = _make_sc_cfgather()(user_ids, item_ids, cfcat)
    return _tc_score(ucf_g, icf_g, uprof_g, icont_g,
                     proj_W, proj_b.reshape(1, CD), ln_gamma.reshape(1, CD),
                     ln_beta.reshape(1, CD))


# speedup vs baseline: 2.9898x; 1.1326x over previous
import functools

import jax
import jax.numpy as jnp
from jax import lax
from jax.experimental import pallas as pl
from jax.experimental.pallas import tpu as pltpu
from jax.experimental.pallas import tpu_sc as plsc

BATCH = 16384
CF_DIM = 64
CD = 256
ALPHA = 0.5

NC = 2
NS = 16
NW = NC * NS
BPW = BATCH // NW

CF_SUB = 128
CF_NSUB = BPW // CF_SUB
PR_SUB = 64
PR_NSUB = BPW // PR_SUB

N_ROWS = 100000
N_PAD = 100096
PREP_R = 5888


def _mesh():
    return plsc.VectorSubcoreMesh(core_axis_name="c", subcore_axis_name="s",
                                  num_cores=NC, num_subcores=NS)


@functools.cache
def _make_sc_cfgather():
    @functools.partial(
        pl.kernel,
        out_type=[
            jax.ShapeDtypeStruct((BATCH, 2 * CF_DIM), jnp.float32),
            jax.ShapeDtypeStruct((BATCH, 2 * CF_DIM), jnp.float32),
        ],
        mesh=_mesh(),
        scratch_types=[
            pltpu.VMEM((BPW,), jnp.int32),
            pltpu.VMEM((BPW,), jnp.int32),
            pltpu.VMEM((2, CF_SUB, 2 * CF_DIM), jnp.float32),
            pltpu.VMEM((2, CF_SUB, 2 * CF_DIM), jnp.float32),
            pltpu.SemaphoreType.DMA,
            pltpu.SemaphoreType.DMA,
            pltpu.SemaphoreType.DMA,
            pltpu.SemaphoreType.DMA,
        ],
    )
    def _sc_cfgather(uids, iids, cfcat, dep, ucf_out, icf_out,
                     uid_v, iid_v, ubuf, ibuf, su0, su1, si0, si1):
        del dep
        wid = lax.axis_index("s") * NC + lax.axis_index("c")
        base = wid * BPW
        pltpu.sync_copy(uids.at[pl.ds(base, BPW)], uid_v)
        pltpu.sync_copy(iids.at[pl.ds(base, BPW)], iid_v)
        sem_u = (su0, su1)
        sem_i = (si0, si1)

        def fire(c):
            o = c * CF_SUB
            s = c % 2
            cu = pltpu.async_copy(cfcat.at[uid_v.at[pl.ds(o, CF_SUB)]],
                                  ubuf.at[s], sem_u[s])
            ci = pltpu.async_copy(cfcat.at[iid_v.at[pl.ds(o, CF_SUB)]],
                                  ibuf.at[s], sem_i[s])
            return cu, ci

        pend = fire(0)
        for c in range(CF_NSUB):
            cu, ci = pend
            if c + 1 < CF_NSUB:
                nxt = fire(c + 1)
            cu.wait()
            ci.wait()
            o = c * CF_SUB
            s = c % 2
            pltpu.sync_copy(ubuf.at[s], ucf_out.at[pl.ds(base + o, CF_SUB)])
            pltpu.sync_copy(ibuf.at[s], icf_out.at[pl.ds(base + o, CF_SUB)])
            if c + 1 < CF_NSUB:
                pend = nxt

    return _sc_cfgather


@functools.cache
def _make_sc_gather():
    @functools.partial(
        pl.kernel,
        out_type=[
            jax.ShapeDtypeStruct((BATCH, CD), jnp.float32),
            jax.ShapeDtypeStruct((BATCH, CD), jnp.float32),
        ],
        mesh=_mesh(),
        scratch_types=[
            pltpu.VMEM((BPW,), jnp.int32),
            pltpu.VMEM((BPW,), jnp.int32),
            pltpu.VMEM((2, PR_SUB, CD), jnp.float32),
            pltpu.VMEM((2, PR_SUB, CD), jnp.float32),
            pltpu.SemaphoreType.DMA,
            pltpu.SemaphoreType.DMA,
            pltpu.SemaphoreType.DMA,
            pltpu.SemaphoreType.DMA,
        ],
    )
    def _sc_gather(uids, iids, uprof, icont,
                   uprof_out, icont_out,
                   uid_v, iid_v, ubuf, ibuf, su0, su1, si0, si1):
        wid = lax.axis_index("s") * NC + lax.axis_index("c")
        base = wid * BPW
        pltpu.sync_copy(uids.at[pl.ds(base, BPW)], uid_v)
        pltpu.sync_copy(iids.at[pl.ds(base, BPW)], iid_v)
        sem_u = (su0, su1)
        sem_i = (si0, si1)

        def fire(c):
            o = c * PR_SUB
            s = c % 2
            cu = pltpu.async_copy(uprof.at[uid_v.at[pl.ds(o, PR_SUB)]],
                                  ubuf.at[s], sem_u[s])
            ci = pltpu.async_copy(icont.at[iid_v.at[pl.ds(o, PR_SUB)]],
                                  ibuf.at[s], sem_i[s])
            return cu, ci

        pend = fire(0)
        for c in range(PR_NSUB):
            cu, ci = pend
            if c + 1 < PR_NSUB:
                nxt = fire(c + 1)
            cu.wait()
            ci.wait()
            o = c * PR_SUB
            s = c % 2
            pltpu.sync_copy(ubuf.at[s], uprof_out.at[pl.ds(base + o, PR_SUB)])
            pltpu.sync_copy(ibuf.at[s], icont_out.at[pl.ds(base + o, PR_SUB)])
            if c + 1 < PR_NSUB:
                pend = nxt

    return _sc_gather


def _tc_prep_body(ut_ref, it_ref, out_ref):
    out_ref[...] = jnp.concatenate([ut_ref[...].T, it_ref[...].T], axis=1)


_tc_prep = pl.pallas_call(
    _tc_prep_body,
    grid=(N_PAD // PREP_R,),
    in_specs=[
        pl.BlockSpec((CF_DIM, PREP_R), lambda i: (0, i)),
        pl.BlockSpec((CF_DIM, PREP_R), lambda i: (0, i)),
    ],
    out_specs=pl.BlockSpec((PREP_R, 2 * CF_DIM), lambda i: (i, 0)),
    out_shape=jax.ShapeDtypeStruct((N_PAD, 2 * CF_DIM), jnp.float32),
)


BLK = 1024


def _tc_body(ucf_ref, icf_ref, uprof_ref, icont_ref, w_ref, b_ref, g_ref,
             beta_ref, out_ref):
    u = uprof_ref[...]
    h = jnp.dot(u, w_ref[...], preferred_element_type=jnp.float32)
    h = h + b_ref[...]
    mu = jnp.mean(h, axis=1, keepdims=True)
    var = jnp.mean((h - mu) * (h - mu), axis=1, keepdims=True)
    hn = (h - mu) * lax.rsqrt(var + 1e-5) * g_ref[...] + beta_ref[...]
    hg = hn * 0.5 * (1.0 + lax.erf(hn * 0.7071067811865476))
    content = jnp.sum(hg * icont_ref[...], axis=1)
    cf = jnp.sum(ucf_ref[:, :CF_DIM] * icf_ref[:, CF_DIM:], axis=1)
    out_ref[...] = ALPHA * cf + (1.0 - ALPHA) * content


_tc_score = pl.pallas_call(
    _tc_body,
    grid=(BATCH // BLK,),
    in_specs=[
        pl.BlockSpec((BLK, 2 * CF_DIM), lambda i: (i, 0)),
        pl.BlockSpec((BLK, 2 * CF_DIM), lambda i: (i, 0)),
        pl.BlockSpec((BLK, CD), lambda i: (i, 0)),
        pl.BlockSpec((BLK, CD), lambda i: (i, 0)),
        pl.BlockSpec((CD, CD), lambda i: (0, 0)),
        pl.BlockSpec((1, CD), lambda i: (0, 0)),
        pl.BlockSpec((1, CD), lambda i: (0, 0)),
        pl.BlockSpec((1, CD), lambda i: (0, 0)),
    ],
    out_specs=pl.BlockSpec((BLK,), lambda i: (i,)),
    out_shape=jax.ShapeDtypeStruct((BATCH,), jnp.float32),
)


def kernel(user_ids, item_ids, user_cf_weight, item_cf_weight,
           raw_user_profiles, article_content_embeddings,
           proj_W, proj_b, ln_gamma, ln_beta):
    uprof_g, icont_g = _make_sc_gather()(
        user_ids, item_ids, raw_user_profiles, article_content_embeddings)
    cfcat = _tc_prep(user_cf_weight.T, item_cf_weight.T)
    ucf_g, icf_g = _make_sc_cfgather()(user_ids, item_ids, cfcat, uprof_g)
    return _tc_score(ucf_g, icf_g, uprof_g, icont_g,
                     proj_W, proj_b.reshape(1, CD), ln_gamma.reshape(1, CD),
                     ln_beta.reshape(1, CD))
